# baseline (speedup 1.0000x reference)
import functools

import jax
import jax.numpy as jnp
from jax.experimental import pallas as pl
from jax.experimental.pallas import tpu as pltpu

_LANES = 128
_RB = 600


def _body(obj_ref, dist_ref, idx_ref, out_ref, *, scale):
    i = pl.program_id(0)

    @pl.when(i == 0)
    def _():
        out_ref[0, 0] = 0.0

    obj = obj_ref[...]
    l_ids = jax.lax.broadcasted_iota(jnp.int32, (3 * _LANES, _LANES), 0)
    p_ids = jax.lax.broadcasted_iota(jnp.int32, (3 * _LANES, _LANES), 1)
    w = (l_ids // 3 == p_ids).astype(jnp.float32)
    s = jnp.dot(obj, w, preferred_element_type=jnp.float32)

    idxf = idx_ref[...].astype(jnp.float32)
    t = idxf * (3.0 * idxf - s)
    contrib = jnp.where(t > 0.0, dist_ref[...], 0.0)
    out_ref[0, 0] += jnp.sum(contrib) * scale


def kernel(hand_xyz, hand_face, obj_xyz, nn_dist, nn_idx):
    del hand_face
    bsz = hand_xyz.shape[0]
    no = obj_xyz.shape[1]
    n = bsz * no
    rows = n // _LANES

    obj_r = obj_xyz.reshape(rows, 3 * _LANES)
    dist_r = nn_dist.reshape(rows, _LANES)
    idx_r = nn_idx.reshape(rows, _LANES)

    out = pl.pallas_call(
        functools.partial(_body, scale=100.0 / bsz),
        grid=(rows // _RB,),
        in_specs=[
            pl.BlockSpec((_RB, 3 * _LANES), lambda i: (i, 0)),
            pl.BlockSpec((_RB, _LANES), lambda i: (i, 0)),
            pl.BlockSpec((_RB, _LANES), lambda i: (i, 0)),
        ],
        out_specs=pl.BlockSpec(
            (1, 1), lambda i: (0, 0), memory_space=pltpu.SMEM
        ),
        out_shape=jax.ShapeDtypeStruct((1, 1), jnp.float32),
    )(obj_r, dist_r, idx_r)
    return out[0, 0]


# speedup vs baseline: 225.7370x; 225.7370x over previous
import functools

import jax
import jax.numpy as jnp
from jax.experimental import pallas as pl
from jax.experimental.pallas import tpu as pltpu

_NOB = 120


def _body(obj_ref, dist_ref, idx_ref, out_ref, *, scale):
    i = pl.program_id(0)

    @pl.when(i == 0)
    def _():
        out_ref[0, 0] = 0.0

    s = obj_ref[0] + obj_ref[1] + obj_ref[2]
    idxf = idx_ref[...].astype(jnp.float32)
    t = idxf * (3.0 * idxf - s)
    contrib = jnp.where(t > 0.0, dist_ref[...], 0.0)
    out_ref[0, 0] += jnp.sum(contrib) * scale


def kernel(hand_xyz, hand_face, obj_xyz, nn_dist, nn_idx):
    del hand_face
    bsz = hand_xyz.shape[0]
    no = obj_xyz.shape[1]

    obj_t = jnp.transpose(obj_xyz, (2, 1, 0))
    dist_t = nn_dist.T
    idx_t = nn_idx.T

    out = pl.pallas_call(
        functools.partial(_body, scale=100.0 / bsz),
        grid=(no // _NOB,),
        in_specs=[
            pl.BlockSpec((3, _NOB, bsz), lambda i: (0, i, 0)),
            pl.BlockSpec((_NOB, bsz), lambda i: (i, 0)),
            pl.BlockSpec((_NOB, bsz), lambda i: (i, 0)),
        ],
        out_specs=pl.BlockSpec(
            (1, 1), lambda i: (0, 0), memory_space=pltpu.SMEM
        ),
        out_shape=jax.ShapeDtypeStruct((1, 1), jnp.float32),
    )(obj_t, dist_t, idx_t)
    return out[0, 0]


# speedup vs baseline: 303.2889x; 1.3435x over previous
import functools

import jax
import jax.numpy as jnp
from jax.experimental import pallas as pl
from jax.experimental.pallas import tpu as pltpu

_NOB = 200


def _body(obj_ref, dist_ref, idx_ref, out_ref, *, scale):
    i = pl.program_id(0)

    @pl.when(i == 0)
    def _():
        out_ref[0, 0] = 0.0

    s = obj_ref[0] + obj_ref[1] + obj_ref[2]
    idxf = idx_ref[...].astype(jnp.float32)
    t = idxf * (3.0 * idxf - s)
    contrib = jnp.where(t > 0.0, dist_ref[...], 0.0)
    out_ref[0, 0] += jnp.sum(contrib) * scale


def kernel(hand_xyz, hand_face, obj_xyz, nn_dist, nn_idx):
    del hand_face
    bsz = hand_xyz.shape[0]
    no = obj_xyz.shape[1]

    obj_t = jnp.transpose(obj_xyz, (2, 1, 0))
    dist_t = nn_dist.T
    idx_t = nn_idx.T

    out = pl.pallas_call(
        functools.partial(_body, scale=100.0 / bsz),
        grid=(no // _NOB,),
        in_specs=[
            pl.BlockSpec((3, _NOB, bsz), lambda i: (0, i, 0)),
            pl.BlockSpec((_NOB, bsz), lambda i: (i, 0)),
            pl.BlockSpec((_NOB, bsz), lambda i: (i, 0)),
        ],
        out_specs=pl.BlockSpec(
            (1, 1), lambda i: (0, 0), memory_space=pltpu.SMEM
        ),
        out_shape=jax.ShapeDtypeStruct((1, 1), jnp.float32),
    )(obj_t, dist_t, idx_t)
    return out[0, 0]


# speedup vs baseline: 447.8605x; 1.4767x over previous
import functools

import jax
import jax.numpy as jnp
from jax.experimental import pallas as pl
from jax.experimental.pallas import tpu as pltpu

_NOB = 600


def _body(obj_ref, dist_ref, idx_ref, out_ref, *, scale):
    i = pl.program_id(0)

    @pl.when(i == 0)
    def _():
        out_ref[0, 0] = 0.0

    s = obj_ref[0] + obj_ref[1] + obj_ref[2]
    idxf = idx_ref[...].astype(jnp.float32)
    t = idxf * (3.0 * idxf - s)
    contrib = jnp.where(t > 0.0, dist_ref[...], 0.0)
    out_ref[0, 0] += jnp.sum(contrib) * scale


def kernel(hand_xyz, hand_face, obj_xyz, nn_dist, nn_idx):
    del hand_face
    bsz = hand_xyz.shape[0]
    no = obj_xyz.shape[1]

    obj_t = jnp.transpose(obj_xyz, (2, 1, 0))
    dist_t = nn_dist.T
    idx_t = nn_idx.T

    out = pl.pallas_call(
        functools.partial(_body, scale=100.0 / bsz),
        grid=(no // _NOB,),
        in_specs=[
            pl.BlockSpec((3, _NOB, bsz), lambda i: (0, i, 0)),
            pl.BlockSpec((_NOB, bsz), lambda i: (i, 0)),
            pl.BlockSpec((_NOB, bsz), lambda i: (i, 0)),
        ],
        out_specs=pl.BlockSpec(
            (1, 1), lambda i: (0, 0), memory_space=pltpu.SMEM
        ),
        out_shape=jax.ShapeDtypeStruct((1, 1), jnp.float32),
    )(obj_t, dist_t, idx_t)
    return out[0, 0]
